# Initial kernel scaffold; baseline (speedup 1.0000x reference)
import jax
import jax.numpy as jnp
from jax.experimental import pallas as pl


def kernel(features, edge_index, adj_values, W_gc1, W_gc2, W_t1, W_t2):
    raise NotImplementedError("write your pallas kernel here")


# speedup vs baseline: 3.2356x; 3.2356x over previous
import functools

import jax
import jax.numpy as jnp
from jax import lax
from jax.experimental import pallas as pl
from jax.experimental.pallas import tpu as pltpu
from jax.experimental.pallas import tpu_sc as plsc

NN = 10000
NP = 10240
DD = 128
HH = 200
HP = 128
CHUNK = 64
SUPER = 32
ROWS_PER_TILE = 320
NSUPER = ROWS_PER_TILE // SUPER
NTILES = 16
EROWS = NTILES * ROWS_PER_TILE
EP = EROWS * CHUNK
NODES_PER_TILE = NP // NTILES
WB = 64
NWB = NODES_PER_TILE // WB

BN = 1024
GRID = NP // BN


def _spmm_body(xcat, src2, dst2, adj1, out, acc, srcb, dstb, adjb,
               g0, g1, g2, g3, s0, s1, gsem, ssem):
    c = lax.axis_index("c")
    s = lax.axis_index("s")

    zero16 = jnp.zeros((16,), jnp.float32)

    def _zrow(r, carry):
        for q in range(HP // 16):
            s0[r, pl.ds(q * 16, 16)] = zero16
        return carry

    lax.fori_loop(0, WB, _zrow, 0)
    nb = s * NODES_PER_TILE

    def _zcopy(k, carry):
        pltpu.sync_copy(s0, acc.at[pl.ds(nb + k * WB, WB)])
        return carry

    lax.fori_loop(0, NWB, _zcopy, 0)
    plsc.subcore_barrier()

    half_off = c * NP

    def _scale_into(g, sbuf, chunk):
        def _grp(k, cz):
            a16 = adjb[pl.ds(chunk * CHUNK + k * 16, 16)]
            for l in range(16):
                spl = a16[l]
                for q in range(HP // 32):
                    vi = g[k * 16 + l, pl.ds(q * 16, 16)]
                    lo = lax.bitcast_convert_type(
                        lax.shift_left(vi, 16), jnp.float32)
                    hi = lax.bitcast_convert_type(
                        lax.bitwise_and(vi, jnp.int32(-65536)),
                        jnp.float32)
                    sbuf[k * 16 + l, pl.ds(q * 16, 16)] = lo * spl
                    sbuf[k * 16 + l, pl.ds(64 + q * 16, 16)] = hi * spl
            return cz
        lax.fori_loop(0, CHUNK // 16, _grp, 0)

    def _super(gg, carry):
        row0 = s * ROWS_PER_TILE + gg * SUPER
        pltpu.sync_copy(src2.at[pl.ds(row0, SUPER)], srcb)
        pltpu.sync_copy(dst2.at[pl.ds(row0, SUPER)], dstb)
        pltpu.sync_copy(adj1.at[pl.ds(row0 * CHUNK, SUPER * CHUNK)], adjb)

        def _shift(r, cy):
            for q in range(CHUNK // 16):
                srcb[r, pl.ds(q * 16, 16)] = (
                    srcb[r, pl.ds(q * 16, 16)] + half_off)
            return cy

        lax.fori_loop(0, SUPER, _shift, 0)

        pltpu.async_copy(xcat.at[srcb.at[0]], g0, gsem)
        pltpu.async_copy(xcat.at[srcb.at[1]], g1, gsem)
        pltpu.async_copy(xcat.at[srcb.at[2]], g2, gsem)

        def _quad(j, cy):
            for u, g in ((0, g0), (1, g1), (2, g2), (3, g3)):
                t = 4 * j + u
                sbuf = (s0, s1)[u % 2]
                pltpu.make_async_copy(xcat.at[srcb.at[t]], g, gsem).wait()

                @pl.when(t > 1)
                def _():
                    pltpu.make_async_copy(
                        sbuf, acc.at[dstb.at[t]], ssem).wait()

                _scale_into(g, sbuf, t)
                pltpu.async_copy(sbuf, acc.at[dstb.at[t]], ssem, add=True)

                @pl.when(t + 3 < SUPER)
                def _():
                    nxt = (g0, g1, g2, g3)[(u + 3) % 4]
                    pltpu.async_copy(xcat.at[srcb.at[t + 3]], nxt, gsem)
            return cy

        lax.fori_loop(0, SUPER // 4, _quad, 0)
        pltpu.make_async_copy(s0, acc.at[dstb.at[0]], ssem).wait()
        pltpu.make_async_copy(s1, acc.at[dstb.at[0]], ssem).wait()
        return carry

    lax.fori_loop(0, NSUPER, _super, 0)
    plsc.subcore_barrier()

    def _wb(k, carry):
        pltpu.sync_copy(acc.at[pl.ds(nb + k * WB, WB)], s0)
        pltpu.sync_copy(s0, out.at[pl.ds(half_off + nb + k * WB, WB)])
        return carry

    lax.fori_loop(0, NWB, _wb, 0)


_spmm = pl.kernel(
    _spmm_body,
    out_type=jax.ShapeDtypeStruct((2 * NP, HP), jnp.float32),
    mesh=plsc.VectorSubcoreMesh(
        core_axis_name="c", subcore_axis_name="s",
        num_cores=2, num_subcores=NTILES),
    compiler_params=pltpu.CompilerParams(
        needs_layout_passes=False, use_tc_tiling_on_sc=False),
    scratch_types=[
        pltpu.VMEM_SHARED((NP, HP), jnp.float32),
        pltpu.VMEM((SUPER, CHUNK), jnp.int32),
        pltpu.VMEM((SUPER, CHUNK), jnp.int32),
        pltpu.VMEM((SUPER * CHUNK,), jnp.float32),
        pltpu.VMEM((CHUNK, HP // 2), jnp.int32),
        pltpu.VMEM((CHUNK, HP // 2), jnp.int32),
        pltpu.VMEM((CHUNK, HP // 2), jnp.int32),
        pltpu.VMEM((CHUNK, HP // 2), jnp.int32),
        pltpu.VMEM((CHUNK, HP), jnp.float32),
        pltpu.VMEM((CHUNK, HP), jnp.float32),
        pltpu.SemaphoreType.DMA,
        pltpu.SemaphoreType.DMA,
    ],
)


def _pack_half(mh):
    lo = lax.bitcast_convert_type(
        mh[:, :HP // 2].astype(jnp.bfloat16),
        jnp.uint16).astype(jnp.uint32)
    hi = lax.bitcast_convert_type(
        mh[:, HP // 2:].astype(jnp.bfloat16),
        jnp.uint16).astype(jnp.uint32)
    return lax.bitcast_convert_type(
        lo | (hi << jnp.uint32(16)), jnp.int32)


def _mm1_body(f_ref, wp1_ref, wpre1_ref, wpre2_ref, xc_ref, fp1_ref, fp2_ref):
    f = f_ref[...]
    m = jnp.dot(f, wp1_ref[...], preferred_element_type=jnp.float32)
    xc_ref[0] = _pack_half(m[:, :HP])
    xc_ref[1] = _pack_half(m[:, HP:])
    fp1_ref[...] = jnp.dot(f, wpre1_ref[...],
                           preferred_element_type=jnp.float32)
    fp2_ref[...] = jnp.dot(f, wpre2_ref[...],
                           preferred_element_type=jnp.float32)


_mm1 = pl.pallas_call(
    _mm1_body,
    grid=(GRID,),
    in_specs=[
        pl.BlockSpec((BN, DD), lambda i: (i, 0)),
        pl.BlockSpec((DD, 2 * HP), lambda i: (0, 0)),
        pl.BlockSpec((DD, HH), lambda i: (0, 0)),
        pl.BlockSpec((DD, HH), lambda i: (0, 0)),
    ],
    out_specs=[
        pl.BlockSpec((2, BN, HP // 2), lambda i: (0, i, 0)),
        pl.BlockSpec((BN, HH), lambda i: (i, 0)),
        pl.BlockSpec((BN, HH), lambda i: (i, 0)),
    ],
    out_shape=[
        jax.ShapeDtypeStruct((2, NP, HP // 2), jnp.int32),
        jax.ShapeDtypeStruct((NP, HH), jnp.float32),
        jax.ShapeDtypeStruct((NP, HH), jnp.float32),
    ],
)


def _mm2_body(p_ref, fp1_ref, wt1_ref, wgc2_ref, h1_ref, xc2_ref):
    t = (jnp.dot(p_ref[0], wt1_ref[0], preferred_element_type=jnp.float32)
         + jnp.dot(p_ref[1], wt1_ref[1], preferred_element_type=jnp.float32)
         + fp1_ref[...])
    h1 = jnp.maximum(t, 0.0)
    h1_ref[...] = h1
    m = jnp.dot(h1, wgc2_ref[...], preferred_element_type=jnp.float32)
    xc2_ref[0] = _pack_half(m[:, :HP])
    xc2_ref[1] = _pack_half(m[:, HP:])


_mm2 = pl.pallas_call(
    _mm2_body,
    grid=(GRID,),
    in_specs=[
        pl.BlockSpec((2, BN, HP), lambda i: (0, i, 0)),
        pl.BlockSpec((BN, HH), lambda i: (i, 0)),
        pl.BlockSpec((2, HP, HH), lambda i: (0, 0, 0)),
        pl.BlockSpec((HH, 2 * HP), lambda i: (0, 0)),
    ],
    out_specs=[
        pl.BlockSpec((BN, HH), lambda i: (i, 0)),
        pl.BlockSpec((2, BN, HP // 2), lambda i: (0, i, 0)),
    ],
    out_shape=[
        jax.ShapeDtypeStruct((NP, HH), jnp.float32),
        jax.ShapeDtypeStruct((2, NP, HP // 2), jnp.int32),
    ],
)


def _mm3_body(q_ref, fp2_ref, wt2_ref, h2_ref):
    t = (jnp.dot(q_ref[0], wt2_ref[0], preferred_element_type=jnp.float32)
         + jnp.dot(q_ref[1], wt2_ref[1], preferred_element_type=jnp.float32)
         + fp2_ref[...])
    h2_ref[...] = jnp.maximum(t, 0.0)


_mm3 = pl.pallas_call(
    _mm3_body,
    grid=(GRID,),
    in_specs=[
        pl.BlockSpec((2, BN, HP), lambda i: (0, i, 0)),
        pl.BlockSpec((BN, HH), lambda i: (i, 0)),
        pl.BlockSpec((2, HP, HH), lambda i: (0, 0, 0)),
    ],
    out_specs=pl.BlockSpec((BN, HH), lambda i: (i, 0)),
    out_shape=jax.ShapeDtypeStruct((NP, HH), jnp.float32),
)


def _pad_cols(w):
    z = jnp.zeros((w.shape[0], HP - 100), w.dtype)
    return jnp.concatenate([w[:, :100], z, w[:, 100:200], z], axis=1)


def _stack_rows(w):
    z = jnp.zeros((HP - 100, w.shape[1]), w.dtype)
    return jnp.stack([jnp.concatenate([w[:100], z]),
                      jnp.concatenate([w[100:200], z])])


def kernel(features, edge_index, adj_values, W_gc1, W_gc2, W_t1, W_t2):
    src = edge_index[0]
    dst = edge_index[1]
    e = src.shape[0]
    pad = EP - e
    src2 = jnp.concatenate(
        [src, jnp.zeros((pad,), jnp.int32)]).reshape(EROWS, CHUNK)
    dst2 = jnp.concatenate(
        [dst, jnp.zeros((pad,), jnp.int32)]).reshape(EROWS, CHUNK)
    adj1 = jnp.concatenate(
        [adj_values, jnp.zeros((pad,), jnp.float32)])

    wp1 = _pad_cols(W_gc1)
    wgc2p = _pad_cols(W_gc2)
    wt1s = _stack_rows(W_t1)
    wt2s = _stack_rows(W_t2)
    wpre1 = W_t1[HH:]
    wpre2 = W_t2[HH:]

    fpad = jnp.pad(features, ((0, NP - NN), (0, 0)))
    xc, fp1, fp2 = _mm1(fpad, wp1, wpre1, wpre2)
    p = _spmm(xc.reshape(2 * NP, HP // 2), src2, dst2, adj1)
    h1, xc2 = _mm2(p.reshape(2, NP, HP), fp1, wt1s, wgc2p)
    q = _spmm(xc2.reshape(2 * NP, HP // 2), src2, dst2, adj1)
    h2 = _mm3(q.reshape(2, NP, HP), fp2, wt2s)
    return (h1[:NN], h2[:NN])
